# baseline (speedup 1.0000x reference)
import functools

import jax
import jax.numpy as jnp
from jax.experimental import pallas as pl
from jax.experimental.pallas import tpu as pltpu

_BIG = 1e9
_LOG2E = 1.4426950408889634
_LOG2E2 = _LOG2E * _LOG2E


def _min2_tree(u):
    w = u.shape[1]
    h = w // 2
    a, b = u[:, :h], u[:, h:]
    m1 = jnp.minimum(a, b)
    m2 = jnp.maximum(a, b)
    w = h
    while w > 1:
        h = w // 2
        a1, b1 = m1[:, :h], m1[:, h:]
        a2, b2 = m2[:, :h], m2[:, h:]
        m1, m2 = (
            jnp.minimum(a1, b1),
            jnp.minimum(jnp.maximum(a1, b1), jnp.minimum(a2, b2)),
        )
        w = h
    return m1, m2


def _process(buf_ref, blk, valid, pen_ref, k):
    rpb = pen_ref.shape[0]
    buf_ref[:, pl.ds(blk * rpb, rpb)] += pen_ref[:]
    u = jnp.maximum(buf_ref[:, :], 1e-20)
    y = jax.lax.rsqrt(u)
    s = jnp.exp2(-(u * y))
    denom = jnp.sum(s, axis=1, keepdims=True)
    m1q, m2q = _min2_tree(u)
    loss_rows = (jnp.sqrt(m1q) + jnp.sqrt(m2q)) * (1.0 / (k * _LOG2E)) \
        + jnp.log(denom)
    part = jnp.sum(loss_rows)[None, None]
    return jnp.where(valid, part, 0.0)


def _knn_loss_step(x_ref, xt_ref, pen_ref, out_ref,
                   x_aug_ref, xt_aug_ref, bufa_ref, bufb_ref,
                   *, k, rows_per_blk, nblk):
    t = pl.program_id(0)
    dim = xt_ref.shape[0]
    rpb = rows_per_blk

    @pl.when(t == 0)
    def _init():
        xt = xt_ref[:]
        sqa = jnp.sum(xt * xt, axis=0, keepdims=True)
        xt_aug_ref[0:dim, :] = xt
        xt_aug_ref[dim:dim + 1, :] = sqa
        xt_aug_ref[dim + 1:dim + 2, :] = jnp.ones_like(sqa)
        x = x_ref[:]
        sq = jnp.sum(x * x, axis=1, keepdims=True)
        x_aug_ref[:, 0:dim] = x * (-2.0 * _LOG2E2)
        x_aug_ref[:, dim:dim + 1] = jnp.full_like(sq, _LOG2E2)
        x_aug_ref[:, dim + 1:dim + 2] = sq * _LOG2E2
        out_ref[:, :] = jnp.zeros((1, 1), jnp.float32)
        bufb_ref[:, :] = jax.lax.dot_general(
            x_aug_ref[0:rpb, :], xt_aug_ref[:], (((1,), (0,)), ((), ())),
            preferred_element_type=jnp.float32)

    dn = (((1,), (0,)), ((), ()))
    blk_a = 2 * t + 1
    blk_b_prev = 2 * t
    blk_b_new = jnp.minimum(2 * t + 2, nblk - 1)

    bufa_ref[:, :] = jax.lax.dot_general(
        x_aug_ref[pl.ds(blk_a * rpb, rpb), :], xt_aug_ref[:], dn,
        preferred_element_type=jnp.float32)
    acc = _process(bufb_ref, blk_b_prev, True, pen_ref, k)
    bufb_ref[:, :] = jax.lax.dot_general(
        x_aug_ref[pl.ds(blk_b_new * rpb, rpb), :], xt_aug_ref[:], dn,
        preferred_element_type=jnp.float32)
    acc += _process(bufa_ref, blk_a, True, pen_ref, k)

    out_ref[:, :] += acc


def kernel(x):
    n, d = x.shape
    rows_per_blk = 256
    nblk = n // rows_per_blk
    pen = _BIG * jnp.eye(rows_per_blk, dtype=jnp.float32)
    out = pl.pallas_call(
        functools.partial(_knn_loss_step, k=2, rows_per_blk=rows_per_blk,
                          nblk=nblk),
        grid=(nblk // 2,),
        in_specs=[
            pl.BlockSpec((n, d), lambda t: (0, 0)),
            pl.BlockSpec((d, n), lambda t: (0, 0)),
            pl.BlockSpec((rows_per_blk, rows_per_blk), lambda t: (0, 0)),
        ],
        out_specs=pl.BlockSpec((1, 1), lambda t: (0, 0)),
        out_shape=jax.ShapeDtypeStruct((1, 1), jnp.float32),
        scratch_shapes=[
            pltpu.VMEM((n, d + 2), jnp.float32),
            pltpu.VMEM((d + 2, n), jnp.float32),
            pltpu.VMEM((rows_per_blk, n), jnp.float32),
            pltpu.VMEM((rows_per_blk, n), jnp.float32),
        ],
    )(x, x.T, pen)
    return out[0, 0] / n


# speedup vs baseline: 1.0251x; 1.0251x over previous
import functools

import jax
import jax.numpy as jnp
from jax.experimental import pallas as pl
from jax.experimental.pallas import tpu as pltpu

_BIG = 1e9
_LOG2E = 1.4426950408889634
_LOG2E2 = _LOG2E * _LOG2E


def _min2_tree(u):
    w = u.shape[1]
    h = w // 2
    a, b = u[:, :h], u[:, h:]
    m1 = jnp.minimum(a, b)
    m2 = jnp.maximum(a, b)
    w = h
    while w > 1:
        h = w // 2
        a1, b1 = m1[:, :h], m1[:, h:]
        a2, b2 = m2[:, :h], m2[:, h:]
        m1, m2 = (
            jnp.minimum(a1, b1),
            jnp.minimum(jnp.maximum(a1, b1), jnp.minimum(a2, b2)),
        )
        w = h
    return m1, m2


def _process(buf_ref, blk, valid, pen_ref, k):
    rpb = pen_ref.shape[0]
    buf_ref[:, pl.ds(blk * rpb, rpb)] += pen_ref[:]
    u = jnp.maximum(buf_ref[:, :], 1e-20)
    y = jax.lax.rsqrt(u)
    s = jnp.exp2(-(u * y))
    denom = jnp.sum(s, axis=1, keepdims=True)
    m1q, m2q = _min2_tree(u)
    loss_rows = (jnp.sqrt(m1q) + jnp.sqrt(m2q)) * (1.0 / (k * _LOG2E)) \
        + jnp.log(denom)
    part = jnp.sum(loss_rows)[None, None]
    return jnp.where(valid, part, 0.0)


def _knn_loss_step(x_ref, xt_ref, pen_ref, out_ref,
                   x_aug_ref, xt_aug_ref, bufa_ref, bufb_ref,
                   *, k, rows_per_blk, nblk):
    t = pl.program_id(0)
    dim = xt_ref.shape[0]
    rpb = rows_per_blk

    @pl.when(t == 0)
    def _init():
        xt = xt_ref[:]
        sqa = jnp.sum(xt * xt, axis=0, keepdims=True)
        xt_aug_ref[0:dim, :] = xt
        xt_aug_ref[dim:dim + 1, :] = sqa
        xt_aug_ref[dim + 1:dim + 2, :] = jnp.ones_like(sqa)
        x = x_ref[:]
        sq = jnp.sum(x * x, axis=1, keepdims=True)
        x_aug_ref[:, 0:dim] = x * (-2.0 * _LOG2E2)
        x_aug_ref[:, dim:dim + 1] = jnp.full_like(sq, _LOG2E2)
        x_aug_ref[:, dim + 1:dim + 2] = sq * _LOG2E2
        out_ref[:, :] = jnp.zeros((1, 1), jnp.float32)
        bufb_ref[:, :] = jax.lax.dot_general(
            x_aug_ref[0:rpb, :], xt_aug_ref[:], (((1,), (0,)), ((), ())),
            preferred_element_type=jnp.float32)

    dn = (((1,), (0,)), ((), ()))
    blk_a = 2 * t + 1
    blk_b_prev = 2 * t
    blk_b_new = jnp.minimum(2 * t + 2, nblk - 1)

    bufa_ref[:, :] = jax.lax.dot_general(
        x_aug_ref[pl.ds(blk_a * rpb, rpb), :], xt_aug_ref[:], dn,
        preferred_element_type=jnp.float32)
    acc = _process(bufb_ref, blk_b_prev, True, pen_ref, k)
    bufb_ref[:, :] = jax.lax.dot_general(
        x_aug_ref[pl.ds(blk_b_new * rpb, rpb), :], xt_aug_ref[:], dn,
        preferred_element_type=jnp.float32)
    acc += _process(bufa_ref, blk_a, True, pen_ref, k)

    out_ref[:, :] += acc


def kernel(x):
    n, d = x.shape
    rows_per_blk = 512
    nblk = n // rows_per_blk
    pen = _BIG * jnp.eye(rows_per_blk, dtype=jnp.float32)
    out = pl.pallas_call(
        functools.partial(_knn_loss_step, k=2, rows_per_blk=rows_per_blk,
                          nblk=nblk),
        grid=(nblk // 2,),
        in_specs=[
            pl.BlockSpec((n, d), lambda t: (0, 0)),
            pl.BlockSpec((d, n), lambda t: (0, 0)),
            pl.BlockSpec((rows_per_blk, rows_per_blk), lambda t: (0, 0)),
        ],
        out_specs=pl.BlockSpec((1, 1), lambda t: (0, 0)),
        out_shape=jax.ShapeDtypeStruct((1, 1), jnp.float32),
        scratch_shapes=[
            pltpu.VMEM((n, d + 2), jnp.float32),
            pltpu.VMEM((d + 2, n), jnp.float32),
            pltpu.VMEM((rows_per_blk, n), jnp.float32),
            pltpu.VMEM((rows_per_blk, n), jnp.float32),
        ],
    )(x, x.T, pen)
    return out[0, 0] / n
